# baseline (speedup 1.0000x reference)
import functools

import jax
import jax.numpy as jnp
from jax import lax
from jax.experimental import pallas as pl
from jax.experimental.pallas import tpu as pltpu
from jax.experimental.pallas import tpu_sc as plsc

_ATTR_INDEX = 8

_NC = 2
_NS = 16
_L = 16
_NW = _NC * _NS
_CH = 128
_NBUF = 3


def _make_sc_kernel(B, D, C):
    n = B // _NW
    nch = n // _CH

    mesh = plsc.VectorSubcoreMesh(core_axis_name="c", subcore_axis_name="s")

    @functools.partial(
        pl.kernel,
        mesh=mesh,
        out_type=jax.ShapeDtypeStruct((B,), jnp.float32),
        compiler_params=pltpu.CompilerParams(needs_layout_passes=False),
        scratch_types=[
            pltpu.VMEM((_NBUF * _CH, D), jnp.float32),
            pltpu.VMEM((_NBUF * _CH, C), jnp.float32),
            pltpu.VMEM((n,), jnp.float32),
            pltpu.SemaphoreType.DMA,
            pltpu.SemaphoreType.DMA,
        ],
    )
    def k(z_hbm, a_hbm, out_hbm, zbuf, abuf, obuf, zsem, asem):
        wid = lax.axis_index("s") * _NC + lax.axis_index("c")
        base = wid * n

        iota = lax.iota(jnp.int32, _L)
        col = jnp.full((_L,), _ATTR_INDEX, jnp.int32)

        def z_copy(ch):
            return pltpu.make_async_copy(
                z_hbm.at[pl.ds(base + ch * _CH, _CH)],
                zbuf.at[pl.ds((ch % _NBUF) * _CH, _CH)],
                zsem,
            )

        def a_copy(ch):
            return pltpu.make_async_copy(
                a_hbm.at[pl.ds(base + ch * _CH, _CH)],
                abuf.at[pl.ds((ch % _NBUF) * _CH, _CH)],
                asem,
            )

        for ch in range(min(_NBUF - 1, nch)):
            z_copy(ch).start()
            a_copy(ch).start()

        for ch in range(nch):
            z_copy(ch).wait()
            a_copy(ch).wait()
            if ch + _NBUF - 1 < nch:
                z_copy(ch + _NBUF - 1).start()
                a_copy(ch + _NBUF - 1).start()
            par = (ch % _NBUF) * _CH

            def extract(j, carry, ch=ch, par=par):
                rows = par + j * _L + iota
                zv = plsc.load_gather(zbuf, [rows, col])
                av = plsc.load_gather(abuf, [rows, zv.astype(jnp.int32)])
                obuf[pl.ds(ch * _CH + j * _L, _L)] = av * jnp.float32(0.999)
                return carry

            lax.fori_loop(0, _CH // _L, extract, 0)

        pltpu.sync_copy(obuf, out_hbm.at[pl.ds(base, n)])

    return k


@jax.jit
def kernel(z, a):
    b, c = a.shape
    return _make_sc_kernel(b, z.shape[1], c)(z, a)


# speedup vs baseline: 1.7386x; 1.7386x over previous
import functools

import jax
import jax.numpy as jnp
from jax import lax
from jax.experimental import pallas as pl
from jax.experimental.pallas import tpu as pltpu
from jax.experimental.pallas import tpu_sc as plsc

_ATTR_INDEX = 8

_NC = 2
_NS = 16
_L = 16
_NW = _NC * _NS
_CH = 128
_NBUF = 3


def _make_sc_kernel(B, D, C):
    n = B // _NW
    nch = n // _CH

    mesh = plsc.VectorSubcoreMesh(core_axis_name="c", subcore_axis_name="s")

    @functools.partial(
        pl.kernel,
        mesh=mesh,
        out_type=jax.ShapeDtypeStruct((B,), jnp.float32),
        compiler_params=pltpu.CompilerParams(needs_layout_passes=False),
        scratch_types=[
            pltpu.VMEM((n,), jnp.int32),
            pltpu.VMEM((_NBUF * _CH, C), jnp.float32),
            pltpu.VMEM((n,), jnp.float32),
            pltpu.SemaphoreType.DMA,
            pltpu.SemaphoreType.DMA,
        ],
    )
    def k(zcol_hbm, a_hbm, out_hbm, zcol, abuf, obuf, zsem, asem):
        wid = lax.axis_index("s") * _NC + lax.axis_index("c")
        base = wid * n

        iota = lax.iota(jnp.int32, _L)

        def a_copy(ch):
            return pltpu.make_async_copy(
                a_hbm.at[pl.ds(base + ch * _CH, _CH)],
                abuf.at[pl.ds((ch % _NBUF) * _CH, _CH)],
                asem,
            )

        for ch in range(min(_NBUF - 1, nch)):
            a_copy(ch).start()

        zc = pltpu.make_async_copy(zcol_hbm.at[pl.ds(base, n)], zcol, zsem)
        zc.start()
        zc.wait()

        for ch in range(nch):
            a_copy(ch).wait()
            if ch + _NBUF - 1 < nch:
                a_copy(ch + _NBUF - 1).start()
            par = (ch % _NBUF) * _CH

            def extract(j, carry, ch=ch, par=par):
                rows = par + j * _L + iota
                idxv = zcol[pl.ds(ch * _CH + j * _L, _L)]
                av = plsc.load_gather(abuf, [rows, idxv])
                obuf[pl.ds(ch * _CH + j * _L, _L)] = av * jnp.float32(0.999)
                return carry

            lax.fori_loop(0, _CH // _L, extract, 0)

        pltpu.sync_copy(obuf, out_hbm.at[pl.ds(base, n)])

    return k


@jax.jit
def kernel(z, a):
    b, c = a.shape
    zcol = z[:, _ATTR_INDEX].astype(jnp.int32)
    return _make_sc_kernel(b, z.shape[1], c)(zcol, a)


# speedup vs baseline: 1.8152x; 1.0441x over previous
import functools

import jax
import jax.numpy as jnp
from jax import lax
from jax.experimental import pallas as pl
from jax.experimental.pallas import tpu as pltpu
from jax.experimental.pallas import tpu_sc as plsc

_ATTR_INDEX = 8

_NC = 2
_NS = 16
_L = 16
_NW = _NC * _NS
_CH = 256
_NBUF = 3


def _make_sc_kernel(B, D, C):
    n = B // _NW
    nch = n // _CH

    mesh = plsc.VectorSubcoreMesh(core_axis_name="c", subcore_axis_name="s")

    @functools.partial(
        pl.kernel,
        mesh=mesh,
        out_type=jax.ShapeDtypeStruct((B,), jnp.float32),
        compiler_params=pltpu.CompilerParams(needs_layout_passes=False),
        scratch_types=[
            pltpu.VMEM((n,), jnp.int32),
            pltpu.VMEM((_NBUF * _CH, C), jnp.float32),
            pltpu.VMEM((n,), jnp.float32),
            pltpu.SemaphoreType.DMA,
            pltpu.SemaphoreType.DMA,
        ],
    )
    def k(zcol_hbm, a_hbm, out_hbm, zcol, abuf, obuf, zsem, asem):
        wid = lax.axis_index("s") * _NC + lax.axis_index("c")
        base = wid * n

        iota = lax.iota(jnp.int32, _L)

        def a_copy(ch):
            return pltpu.make_async_copy(
                a_hbm.at[pl.ds(base + ch * _CH, _CH)],
                abuf.at[pl.ds((ch % _NBUF) * _CH, _CH)],
                asem,
            )

        for ch in range(min(_NBUF - 1, nch)):
            a_copy(ch).start()

        zc = pltpu.make_async_copy(zcol_hbm.at[pl.ds(base, n)], zcol, zsem)
        zc.start()
        zc.wait()

        for ch in range(nch):
            a_copy(ch).wait()
            if ch + _NBUF - 1 < nch:
                a_copy(ch + _NBUF - 1).start()
            par = (ch % _NBUF) * _CH

            def extract(j, carry, ch=ch, par=par):
                rows = par + j * _L + iota
                idxv = zcol[pl.ds(ch * _CH + j * _L, _L)]
                av = plsc.load_gather(abuf, [rows, idxv])
                obuf[pl.ds(ch * _CH + j * _L, _L)] = av * jnp.float32(0.999)
                return carry

            lax.fori_loop(0, _CH // _L, extract, 0)

        pltpu.sync_copy(obuf, out_hbm.at[pl.ds(base, n)])

    return k


@jax.jit
def kernel(z, a):
    b, c = a.shape
    zcol = z[:, _ATTR_INDEX].astype(jnp.int32)
    return _make_sc_kernel(b, z.shape[1], c)(zcol, a)
